# baseline (speedup 1.0000x reference)
import functools

import jax
import jax.numpy as jnp
from jax import lax
from jax.experimental import pallas as pl
from jax.experimental.pallas import tpu as pltpu
from jax.experimental.pallas import tpu_sc as plsc

N_EDGES = 160000
D_FEAT = 256
N_NODES = 10000

NC = 2
NS = 16
CHUNK = 128
NCHUNKS = N_EDGES // CHUNK
CPT = 80
HALF = D_FEAT // NC
N_PAD = 10240
RPT = N_PAD // NS
LAST_RPT = N_NODES - (NS - 1) * RPT
NBUF = 2


def _sc_segment_sum(i, idx2, zrows):
    mesh = plsc.VectorSubcoreMesh(core_axis_name="c", subcore_axis_name="s")

    @functools.partial(
        pl.kernel,
        out_type=jax.ShapeDtypeStruct((N_NODES, D_FEAT), jnp.float32),
        mesh=mesh,
        scratch_types=[
            pltpu.VMEM((CPT, CHUNK), jnp.int32),
            [pltpu.VMEM((CHUNK, HALF), jnp.float32) for _ in range(NBUF)],
            pltpu.VMEM_SHARED((N_PAD, HALF), jnp.float32),
            [pltpu.SemaphoreType.DMA for _ in range(NBUF)],
            [pltpu.SemaphoreType.DMA for _ in range(NBUF)],
            pltpu.SemaphoreType.DMA,
        ],
    )
    def k(i_hbm, idx_hbm, z_hbm, out_hbm, idx_v, bufs, accum, gsems, ssems,
          zsem):
        cc = lax.axis_index("c")
        s = lax.axis_index("s")
        base = s * CPT
        n = jnp.minimum(CPT, NCHUNKS - base)

        def gslice(c):
            return i_hbm.at[pl.ds(c * CHUNK, CHUNK), pl.ds(cc * HALF, HALF)]

        idx_cp = pltpu.async_copy(idx_hbm.at[pl.ds(base, CPT)], idx_v, zsem)
        for b in range(NBUF):
            pltpu.async_copy(gslice(base + b), bufs[b], gsems[b])
        for t in range(RPT // CHUNK):
            pltpu.sync_copy(z_hbm, accum.at[pl.ds(s * RPT + t * CHUNK, CHUNK)])
        idx_cp.wait()
        plsc.subcore_barrier()

        def body(j2, carry):
            for b in range(NBUF):
                j = j2 * NBUF + b
                c = base + j
                ob = (b + 1) % NBUF

                @pl.when(j < n)
                def _():
                    pltpu.make_async_copy(gslice(c), bufs[b], gsems[b]).wait()
                    pltpu.async_copy(bufs[b], accum.at[idx_v.at[j]], ssems[b],
                                     add=True)

                @pl.when((j >= 1) & (j + 1 < n))
                def _():
                    pltpu.make_async_copy(gslice(base), bufs[ob],
                                          ssems[ob]).wait()
                    pltpu.async_copy(gslice(c + 1), bufs[ob], gsems[ob])

            return carry

        lax.fori_loop(0, CPT // NBUF, body, 0)
        for b in range(NBUF):
            pltpu.make_async_copy(gslice(base), bufs[b], ssems[b]).wait()
        plsc.subcore_barrier()

        @pl.when(s < NS - 1)
        def _full_copy():
            pltpu.sync_copy(
                accum.at[pl.ds(s * RPT, RPT)],
                out_hbm.at[pl.ds(s * RPT, RPT), pl.ds(cc * HALF, HALF)],
            )

        @pl.when(s == NS - 1)
        def _last_copy():
            pltpu.sync_copy(
                accum.at[pl.ds((NS - 1) * RPT, LAST_RPT)],
                out_hbm.at[pl.ds((NS - 1) * RPT, LAST_RPT),
                           pl.ds(cc * HALF, HALF)],
            )

    return k(i, idx2, zrows)


@jax.jit
def kernel(i, idx_i):
    pad = NS * CPT * CHUNK - N_EDGES
    idx2 = jnp.pad(idx_i, (0, pad)).reshape(NS * CPT, CHUNK)
    zrows = jnp.zeros((CHUNK, HALF), jnp.float32)
    return _sc_segment_sum(i, idx2, zrows)


# speedup vs baseline: 1.1294x; 1.1294x over previous
import functools

import jax
import jax.numpy as jnp
from jax import lax
from jax.experimental import pallas as pl
from jax.experimental.pallas import tpu as pltpu
from jax.experimental.pallas import tpu_sc as plsc

N_EDGES = 160000
D_FEAT = 256
N_NODES = 10000

NC = 2
NS = 16
CHUNK = 128
NCHUNKS = N_EDGES // CHUNK
CPT = 80
HALF = D_FEAT // NC
N_PAD = 10240
RPT = N_PAD // NS
LAST_RPT = N_NODES - (NS - 1) * RPT
NBUF = 2


def _sc_segment_sum(i, idx2, zrows):
    mesh = plsc.VectorSubcoreMesh(core_axis_name="c", subcore_axis_name="s")

    @functools.partial(
        pl.kernel,
        out_type=jax.ShapeDtypeStruct((N_NODES, D_FEAT), jnp.float32),
        mesh=mesh,
        scratch_types=[
            pltpu.VMEM((CPT, CHUNK), jnp.int32),
            [pltpu.VMEM((CHUNK, HALF), jnp.float32) for _ in range(NBUF)],
            pltpu.VMEM_SHARED((N_PAD, HALF), jnp.float32),
            [pltpu.SemaphoreType.DMA for _ in range(NBUF)],
            [pltpu.SemaphoreType.DMA for _ in range(NBUF)],
            pltpu.SemaphoreType.DMA,
        ],
    )
    def k(i_hbm, idx_hbm, z_hbm, out_hbm, idx_v, bufs, accum, gsems, ssems,
          zsem):
        cc = lax.axis_index("c")
        s = lax.axis_index("s")
        base = s * CPT
        n = jnp.minimum(CPT, NCHUNKS - base)

        def gslice(c):
            return i_hbm.at[pl.ds(c * CHUNK, CHUNK), pl.ds(cc * HALF, HALF)]

        idx_cp = pltpu.async_copy(idx_hbm.at[pl.ds(base, CPT)], idx_v, zsem)
        for b in range(NBUF):
            pltpu.async_copy(gslice(base + b), bufs[b], gsems[b])
        zcps = [
            pltpu.async_copy(z_hbm, accum.at[pl.ds(s * RPT + t * CHUNK, CHUNK)],
                             ssems[0])
            for t in range(RPT // CHUNK)
        ]
        for z in zcps:
            z.wait()
        idx_cp.wait()
        plsc.subcore_barrier()

        def body(j2, carry):
            for b in range(NBUF):
                j = j2 * NBUF + b
                c = base + j

                @pl.when(j < n)
                def _():
                    pltpu.make_async_copy(gslice(c), bufs[b], gsems[b]).wait()
                    pltpu.sync_copy(bufs[b], accum.at[idx_v.at[j]], add=True)

                    @pl.when(j + NBUF < n)
                    def _():
                        pltpu.async_copy(gslice(c + NBUF), bufs[b], gsems[b])

            return carry

        lax.fori_loop(0, CPT // NBUF, body, 0)
        plsc.subcore_barrier()

        @pl.when(s < NS - 1)
        def _full_copy():
            pltpu.sync_copy(
                accum.at[pl.ds(s * RPT, RPT)],
                out_hbm.at[pl.ds(s * RPT, RPT), pl.ds(cc * HALF, HALF)],
            )

        @pl.when(s == NS - 1)
        def _last_copy():
            pltpu.sync_copy(
                accum.at[pl.ds((NS - 1) * RPT, LAST_RPT)],
                out_hbm.at[pl.ds((NS - 1) * RPT, LAST_RPT),
                           pl.ds(cc * HALF, HALF)],
            )

    return k(i, idx2, zrows)


@jax.jit
def kernel(i, idx_i):
    pad = NS * CPT * CHUNK - N_EDGES
    idx2 = jnp.pad(idx_i, (0, pad)).reshape(NS * CPT, CHUNK)
    zrows = jnp.zeros((CHUNK, HALF), jnp.float32)
    return _sc_segment_sum(i, idx2, zrows)
